# Initial kernel scaffold; baseline (speedup 1.0000x reference)
import jax
import jax.numpy as jnp
from jax.experimental import pallas as pl


def kernel(indices, tables):
    raise NotImplementedError("write your pallas kernel here")


# speedup vs baseline: 1.1246x; 1.1246x over previous
import jax
import jax.numpy as jnp
from jax import lax
from jax.experimental import pallas as pl
from jax.experimental.pallas import tpu as pltpu
from jax.experimental.pallas import tpu_sc as plsc

_NUM_FIELDS = 26
_VOCAB = 100000
_EMBED = 16
_BATCH = 16384

_NC, _NS, _LANES = 2, 16, 16
_NW = _NC * _NS
_N = _BATCH * _NUM_FIELDS
_NPW = _N // _NW
_CH = 128
_NCH = _NPW // _CH
_SL = _CH // _LANES


def _flatten_idx_chunk(idx_v, fidx_v, chunk):
    lane = lax.iota(jnp.int32, _LANES)
    for c in range(_SL):
        start = chunk * _CH + c * _LANES
        raw = idx_v[pl.ds(start, _LANES)]
        field = (start + lane) % _NUM_FIELDS
        fidx_v[pl.ds(start, _LANES)] = raw + field * _VOCAB


def _body(idx_hbm, tab_hbm, out_hbm, idx_v, fidx_v, rows_v, sem0, sem1):
    wid = lax.axis_index("s") * _NC + lax.axis_index("c")
    base = wid * _NPW
    pltpu.sync_copy(idx_hbm.at[pl.ds(base, _NPW)], idx_v)
    sems = (sem0, sem1)

    def fire(chunk, b):
        _flatten_idx_chunk(idx_v, fidx_v, chunk)
        pltpu.async_copy(
            tab_hbm.at[fidx_v.at[pl.ds(chunk * _CH, _CH)]],
            rows_v.at[b], sems[b])

    def drain(b):
        pltpu.make_async_copy(
            tab_hbm.at[pl.ds(0, _CH)], rows_v.at[b], sems[b]).wait()

    fire(0, 0)

    def group(jj, carry):
        for b in range(2):
            j = jj * 2 + b
            fire(jnp.minimum(j + 1, _NCH - 1), 1 - b)
            drain(b)
            pltpu.sync_copy(rows_v.at[b],
                            out_hbm.at[pl.ds(base + j * _CH, _CH)])
        return carry

    lax.fori_loop(0, _NCH // 2, group, 0)
    drain(0)


_mesh = plsc.VectorSubcoreMesh(
    core_axis_name="c", subcore_axis_name="s",
    num_cores=_NC, num_subcores=_NS)

_launch = pl.kernel(
    _body,
    out_type=jax.ShapeDtypeStruct((_N, _EMBED), jnp.float32),
    mesh=_mesh,
    scratch_types=[
        pltpu.VMEM((_NPW,), jnp.int32),
        pltpu.VMEM((_NPW,), jnp.int32),
        pltpu.VMEM((2, _CH, _EMBED), jnp.float32),
        pltpu.SemaphoreType.DMA,
        pltpu.SemaphoreType.DMA,
    ],
    compiler_params=pltpu.CompilerParams(use_tc_tiling_on_sc=False),
)


@jax.jit
def kernel(indices, tables):
    flat_tab = tables.reshape(_NUM_FIELDS * _VOCAB, _EMBED)
    flat_idx = indices.reshape(_N)
    out = _launch(flat_idx, flat_tab)
    return out.reshape(_BATCH, _NUM_FIELDS * _EMBED)


# speedup vs baseline: 6.4552x; 5.7399x over previous
import jax
import jax.numpy as jnp
from jax import lax
from jax.experimental import pallas as pl
from jax.experimental.pallas import tpu as pltpu
from jax.experimental.pallas import tpu_sc as plsc

_NUM_FIELDS = 26
_VOCAB = 100000
_EMBED = 16
_BATCH = 16384

_NC, _NS, _LANES = 2, 16, 16
_NW = _NC * _NS
_NROWS = _NUM_FIELDS * _EMBED
_RPW = _NROWS // _NW
_HALF = _BATCH // 2
_GROUPS = _HALF // _LANES


def _body(idx_hbm, tab_hbm, out_hbm, row_v, idx_v, out_v, sem):
    wid = lax.axis_index("s") * _NC + lax.axis_index("c")

    def do_row(i, carry):
        r = wid * _RPW + i
        f = r // _EMBED
        e = r % _EMBED
        pltpu.async_copy(tab_hbm.at[f, e], row_v, sem).wait()
        for h in range(2):
            pltpu.sync_copy(idx_hbm.at[f, pl.ds(h * _HALF, _HALF)], idx_v)

            def gather_group(g, carry2):
                b = g * _LANES
                vidx = idx_v[pl.ds(b, _LANES)]
                out_v[pl.ds(b, _LANES)] = plsc.load_gather(row_v, [vidx])
                return carry2

            lax.fori_loop(0, _GROUPS, gather_group, 0)
            pltpu.sync_copy(out_v, out_hbm.at[r, pl.ds(h * _HALF, _HALF)])
        return carry

    lax.fori_loop(0, _RPW, do_row, 0)


_mesh = plsc.VectorSubcoreMesh(
    core_axis_name="c", subcore_axis_name="s",
    num_cores=_NC, num_subcores=_NS)

_launch = pl.kernel(
    _body,
    out_type=jax.ShapeDtypeStruct((_NROWS, _BATCH), jnp.float32),
    mesh=_mesh,
    scratch_types=[
        pltpu.VMEM((_VOCAB,), jnp.float32),
        pltpu.VMEM((_HALF,), jnp.int32),
        pltpu.VMEM((_HALF,), jnp.float32),
        pltpu.SemaphoreType.DMA,
    ],
    compiler_params=pltpu.CompilerParams(needs_layout_passes=False),
)


@jax.jit
def kernel(indices, tables):
    tab_t = tables.transpose(0, 2, 1)
    idx_t = indices.T
    out_t = _launch(idx_t, tab_t)
    return out_t.T


# speedup vs baseline: 8.2436x; 1.2770x over previous
import jax
import jax.numpy as jnp
from jax import lax
from jax.experimental import pallas as pl
from jax.experimental.pallas import tpu as pltpu
from jax.experimental.pallas import tpu_sc as plsc

_NUM_FIELDS = 26
_VOCAB = 100000
_EMBED = 16
_BATCH = 16384

_NC, _NS, _LANES = 2, 16, 16
_NW = _NC * _NS
_NROWS = _NUM_FIELDS * _EMBED
_RPW = _NROWS // _NW
_UNROLL = 8
_GROUPS = _BATCH // _LANES // _UNROLL


def _gather_row(row_v, io_v):
    def step(g, carry):
        base = g * _LANES * _UNROLL
        for u in range(_UNROLL):
            b = base + u * _LANES
            vidx = plsc.bitcast(io_v[pl.ds(b, _LANES)], jnp.int32)
            io_v[pl.ds(b, _LANES)] = plsc.load_gather(row_v, [vidx])
        return carry

    lax.fori_loop(0, _GROUPS, step, 0)


def _body(idx_hbm, tab_hbm, out_hbm, row_v, io_v, sem_r, sem_o):
    wid = lax.axis_index("s") * _NC + lax.axis_index("c")
    r0 = wid * _RPW

    pltpu.async_copy(tab_hbm.at[r0 // _EMBED, r0 % _EMBED], row_v, sem_r)
    pltpu.sync_copy(idx_hbm.at[r0 // _EMBED], io_v)

    def wait_row():
        pltpu.make_async_copy(tab_hbm.at[0, 0], row_v, sem_r).wait()

    def wait_out(r):
        pltpu.make_async_copy(io_v, out_hbm.at[r], sem_o).wait()

    def do_row(i, carry):
        r = r0 + i
        rn = r + 1
        wait_row()
        _gather_row(row_v, io_v)
        pltpu.async_copy(tab_hbm.at[rn // _EMBED, rn % _EMBED], row_v, sem_r)
        pltpu.async_copy(io_v, out_hbm.at[r], sem_o)
        wait_out(r)
        pltpu.sync_copy(idx_hbm.at[rn // _EMBED], io_v)
        return carry

    lax.fori_loop(0, _RPW - 1, do_row, 0)

    r_last = r0 + _RPW - 1
    wait_row()
    _gather_row(row_v, io_v)
    pltpu.async_copy(io_v, out_hbm.at[r_last], sem_o)
    wait_out(r_last)


_mesh = plsc.VectorSubcoreMesh(
    core_axis_name="c", subcore_axis_name="s",
    num_cores=_NC, num_subcores=_NS)

_launch = pl.kernel(
    _body,
    out_type=jax.ShapeDtypeStruct((_NROWS, _BATCH), jnp.float32),
    mesh=_mesh,
    scratch_types=[
        pltpu.VMEM((_VOCAB,), jnp.float32),
        pltpu.VMEM((_BATCH,), jnp.float32),
        pltpu.SemaphoreType.DMA,
        pltpu.SemaphoreType.DMA,
    ],
    compiler_params=pltpu.CompilerParams(needs_layout_passes=False),
)


@jax.jit
def kernel(indices, tables):
    tab_t = tables.transpose(0, 2, 1)
    idx_t = lax.bitcast_convert_type(indices.T, jnp.float32)
    out_t = _launch(idx_t, tab_t)
    return out_t.T


# speedup vs baseline: 10.7918x; 1.3091x over previous
import jax
import jax.numpy as jnp
from jax import lax
from jax.experimental import pallas as pl
from jax.experimental.pallas import tpu as pltpu
from jax.experimental.pallas import tpu_sc as plsc

_NUM_FIELDS = 26
_VOCAB = 100000
_EMBED = 16
_BATCH = 16384

_NC, _NS, _LANES = 2, 16, 16
_NW = _NC * _NS
_NROWS = _NUM_FIELDS * _EMBED
_RPW = _NROWS // _NW
_UNROLL = 8
_GROUPS = _BATCH // _LANES // _UNROLL


def _gather_row(row_v, io_v):
    @plsc.parallel_loop(0, _BATCH, step=_LANES * _UNROLL)
    def step(base):
        for u in range(_UNROLL):
            b = base + u * _LANES
            vidx = plsc.bitcast(io_v[pl.ds(b, _LANES)], jnp.int32)
            io_v[pl.ds(b, _LANES)] = plsc.load_gather(row_v, [vidx])


def _body(idx_hbm, tab_hbm, out_hbm, row_v, io_v, sem_r, sem_o):
    wid = lax.axis_index("s") * _NC + lax.axis_index("c")
    r0 = wid * _RPW

    pltpu.async_copy(tab_hbm.at[r0 // _EMBED, r0 % _EMBED], row_v, sem_r)
    pltpu.sync_copy(idx_hbm.at[r0 // _EMBED], io_v)

    def wait_row():
        pltpu.make_async_copy(tab_hbm.at[0, 0], row_v, sem_r).wait()

    def wait_out(r):
        pltpu.make_async_copy(io_v, out_hbm.at[r], sem_o).wait()

    def do_row(i, carry):
        r = r0 + i
        rn = r + 1
        wait_row()
        _gather_row(row_v, io_v)
        pltpu.async_copy(tab_hbm.at[rn // _EMBED, rn % _EMBED], row_v, sem_r)
        pltpu.async_copy(io_v, out_hbm.at[r], sem_o)
        wait_out(r)
        pltpu.sync_copy(idx_hbm.at[rn // _EMBED], io_v)
        return carry

    lax.fori_loop(0, _RPW - 1, do_row, 0)

    r_last = r0 + _RPW - 1
    wait_row()
    _gather_row(row_v, io_v)
    pltpu.async_copy(io_v, out_hbm.at[r_last], sem_o)
    wait_out(r_last)


_mesh = plsc.VectorSubcoreMesh(
    core_axis_name="c", subcore_axis_name="s",
    num_cores=_NC, num_subcores=_NS)

_launch = pl.kernel(
    _body,
    out_type=jax.ShapeDtypeStruct((_NROWS, _BATCH), jnp.float32),
    mesh=_mesh,
    scratch_types=[
        pltpu.VMEM((_VOCAB,), jnp.float32),
        pltpu.VMEM((_BATCH,), jnp.float32),
        pltpu.SemaphoreType.DMA,
        pltpu.SemaphoreType.DMA,
    ],
    compiler_params=pltpu.CompilerParams(needs_layout_passes=False),
)


@jax.jit
def kernel(indices, tables):
    tab_t = tables.transpose(0, 2, 1)
    idx_t = lax.bitcast_convert_type(indices.T, jnp.float32)
    out_t = _launch(idx_t, tab_t)
    return out_t.T
